# baseline (speedup 1.0000x reference)
import jax
import jax.numpy as jnp
from jax.experimental import pallas as pl

N_NODES = 10000
N_EDGES = 320000
HIDDEN = 128
HEADS = 4
N_GRAPHS = 8


def _apply_lin(p, x):
    return x @ p["w"] + p["b"]


def _apply_mlp(ps, x, act=jax.nn.gelu, final_act=None):
    for i, p in enumerate(ps):
        x = _apply_lin(p, x)
        if i < len(ps) - 1:
            x = act(x)
    if final_act is not None:
        x = final_act(x)
    return x


def _apply_bn(p, x):
    mu = x.mean(0)
    var = x.var(0)
    return (x - mu) / jnp.sqrt(var + 1e-5) * p["g"] + p["b"]


def _segment_softmax(w, seg, n):
    wmax = jax.ops.segment_max(w, seg, num_segments=n)
    wmax = jnp.where(jnp.isfinite(wmax), wmax, 0.0)
    w = w - wmax[seg]
    ew = jnp.exp(w)
    denom = jax.ops.segment_sum(ew, seg, num_segments=n)
    return ew / (denom[seg] + 1e-9)


def kernel(h_V, h_E, params, edge_idx, batch_id):
    src = edge_idx[0]
    dst = edge_idx[1]
    h_V = _apply_lin(params["node_embed"], h_V)
    h_E = _apply_lin(params["edge_embed"], h_E)
    for lp in params["layers"]:
        h_EV = jnp.concatenate([h_E, h_V[src]], axis=-1)
        w = _apply_mlp(lp["att_bias"], jnp.concatenate([h_V[dst], h_EV], axis=-1))
        attend = _segment_softmax(w, dst, N_NODES)
        Vm = _apply_mlp(lp["w_v"], h_EV).reshape(-1, HEADS, HIDDEN // HEADS)
        msg = (attend[:, :, None] * Vm).reshape(-1, HIDDEN)
        agg = jax.ops.segment_sum(msg, dst, num_segments=N_NODES)
        agg = _apply_lin(lp["w_o"], agg)
        h_V = _apply_bn(lp["bn0"], h_V + agg)
        dh = _apply_mlp(lp["dense"], h_V)
        h_V = _apply_bn(lp["bn1"], h_V + dh)
        h_EV2 = jnp.concatenate([h_V[src], h_E, h_V[dst]], axis=-1)
        m = _apply_lin(lp["w13"], jax.nn.gelu(_apply_lin(lp["w12"], jax.nn.gelu(_apply_lin(lp["w11"], h_EV2)))))
        h_E = _apply_bn(lp["bn_e"], h_E + m)
        cnt = jax.ops.segment_sum(jnp.ones((N_NODES,), jnp.float32), batch_id, num_segments=N_GRAPHS)
        c_V = jax.ops.segment_sum(h_V, batch_id, num_segments=N_GRAPHS) / (cnt[:, None] + 1e-9)
        gate = _apply_mlp(lp["ctx_g"], c_V, act=jax.nn.relu, final_act=jax.nn.sigmoid)
        h_V = h_V * gate[batch_id]
    logits = _apply_lin(params["readout"], h_V)
    return jax.nn.log_softmax(logits, axis=-1)


# speedup vs baseline: 2.8909x; 2.8909x over previous
import functools

import jax
import jax.numpy as jnp
from jax import lax
from jax.experimental import pallas as pl
from jax.experimental.pallas import tpu as pltpu
from jax.experimental.pallas import tpu_sc as plsc

N_NODES = 10000
N_EDGES = 320000
HIDDEN = 128
HEADS = 4
N_GRAPHS = 8

_NC = 2
_NS = 16
_NW = _NC * _NS


def _sc_gather(table, idx, chunk=80):
    E = idx.shape[0]
    D = table.shape[1]
    per_w = E // _NW
    assert per_w * _NW == E and per_w % chunk == 0
    nch = per_w // chunk
    mesh = plsc.VectorSubcoreMesh(
        core_axis_name="c", subcore_axis_name="s", num_cores=_NC, num_subcores=_NS
    )

    @functools.partial(
        pl.kernel,
        mesh=mesh,
        out_type=jax.ShapeDtypeStruct((E, D), jnp.float32),
        scratch_types=[
            pltpu.VMEM((chunk,), jnp.int32),
            pltpu.VMEM((chunk, D), jnp.float32),
            pltpu.SemaphoreType.DMA,
        ],
    )
    def k(table_hbm, idx_hbm, out_hbm, idx_v, rows_v, sem):
        wid = lax.axis_index("s") * _NC + lax.axis_index("c")
        base = wid * per_w

        def body(j, carry):
            off = base + j * chunk
            pltpu.sync_copy(idx_hbm.at[pl.ds(off, chunk)], idx_v)
            pltpu.async_copy(table_hbm.at[idx_v], rows_v, sem).wait()
            pltpu.sync_copy(rows_v, out_hbm.at[pl.ds(off, chunk)])
            return carry

        lax.fori_loop(0, nch, body, 0)

    return k(table, idx)


def _sc_scatter_add(vals, idx, n, chunk=80):
    E, D = vals.shape
    per_w = E // _NW
    assert per_w * _NW == E and per_w % chunk == 0
    nch = per_w // chunk
    n_pad = ((n + 8 * _NS - 1) // (8 * _NS)) * (8 * _NS)
    rows_per_sub = n_pad // _NS
    mesh = plsc.VectorSubcoreMesh(
        core_axis_name="c", subcore_axis_name="s", num_cores=_NC, num_subcores=_NS
    )

    @functools.partial(
        pl.kernel,
        mesh=mesh,
        out_type=jax.ShapeDtypeStruct((_NC, n_pad, D), jnp.float32),
        scratch_types=[
            pltpu.VMEM((chunk,), jnp.int32),
            pltpu.VMEM((chunk, D), jnp.float32),
            pltpu.VMEM_SHARED((n_pad, D), jnp.float32),
            pltpu.SemaphoreType.DMA,
        ],
    )
    def k(zeros_hbm, vals_hbm, idx_hbm, out_hbm, idx_v, vals_v, table_sh, sem):
        cid = lax.axis_index("c")
        sid = lax.axis_index("s")
        wid = sid * _NC + cid
        srow = sid * rows_per_sub
        pltpu.sync_copy(
            zeros_hbm.at[pl.ds(srow, rows_per_sub)],
            table_sh.at[pl.ds(srow, rows_per_sub)],
        )
        plsc.subcore_barrier()
        base = wid * per_w

        def body(j, carry):
            off = base + j * chunk
            pltpu.sync_copy(idx_hbm.at[pl.ds(off, chunk)], idx_v)
            pltpu.sync_copy(vals_hbm.at[pl.ds(off, chunk)], vals_v)
            pltpu.sync_copy(vals_v, table_sh.at[idx_v], add=True)
            return carry

        lax.fori_loop(0, nch, body, 0)
        plsc.subcore_barrier()
        pltpu.sync_copy(
            table_sh.at[pl.ds(srow, rows_per_sub)],
            out_hbm.at[cid, pl.ds(srow, rows_per_sub)],
        )

    return k(jnp.zeros((n_pad, D), jnp.float32), vals, idx)[:, :n, :]


def _apply_lin(p, x):
    return x @ p["w"] + p["b"]


def _apply_mlp(ps, x, act=jax.nn.gelu, final_act=None):
    for i, p in enumerate(ps):
        x = _apply_lin(p, x)
        if i < len(ps) - 1:
            x = act(x)
    if final_act is not None:
        x = final_act(x)
    return x


def _apply_bn(p, x):
    mu = x.mean(0)
    var = x.var(0)
    return (x - mu) / jnp.sqrt(var + 1e-5) * p["g"] + p["b"]


def _segment_softmax(w, seg, n):
    wmax = jax.ops.segment_max(w, seg, num_segments=n)
    wmax = jnp.where(jnp.isfinite(wmax), wmax, 0.0)
    w = w - wmax[seg]
    ew = jnp.exp(w)
    denom = jax.ops.segment_sum(ew, seg, num_segments=n)
    return ew / (denom[seg] + 1e-9)


def kernel(h_V, h_E, params, edge_idx, batch_id):
    src = edge_idx[0]
    dst = edge_idx[1]
    h_V = _apply_lin(params["node_embed"], h_V)
    h_E = _apply_lin(params["edge_embed"], h_E)
    for lp in params["layers"]:
        hs = _sc_gather(h_V, src)
        hd = _sc_gather(h_V, dst)
        h_EV = jnp.concatenate([h_E, hs], axis=-1)
        w = _apply_mlp(lp["att_bias"], jnp.concatenate([hd, h_EV], axis=-1))
        ew = jnp.exp(w - jnp.max(w))
        ewrep = jnp.repeat(ew, HIDDEN // HEADS, axis=1)
        denom_rep = _sc_scatter_add(ewrep, dst, N_NODES).sum(0)
        Vm = _apply_mlp(lp["w_v"], h_EV).reshape(-1, HEADS, HIDDEN // HEADS)
        msgu = (ew[:, :, None] * Vm).reshape(-1, HIDDEN)
        agg = _sc_scatter_add(msgu, dst, N_NODES).sum(0)
        agg = agg / (denom_rep + 1e-9)
        agg = _apply_lin(lp["w_o"], agg)
        h_V = _apply_bn(lp["bn0"], h_V + agg)
        dh = _apply_mlp(lp["dense"], h_V)
        h_V = _apply_bn(lp["bn1"], h_V + dh)
        hs2 = _sc_gather(h_V, src)
        hd2 = _sc_gather(h_V, dst)
        h_EV2 = jnp.concatenate([hs2, h_E, hd2], axis=-1)
        m = _apply_lin(lp["w13"], jax.nn.gelu(_apply_lin(lp["w12"], jax.nn.gelu(_apply_lin(lp["w11"], h_EV2)))))
        h_E = _apply_bn(lp["bn_e"], h_E + m)
        cnt = jax.ops.segment_sum(jnp.ones((N_NODES,), jnp.float32), batch_id, num_segments=N_GRAPHS)
        c_V = jax.ops.segment_sum(h_V, batch_id, num_segments=N_GRAPHS) / (cnt[:, None] + 1e-9)
        gate = _apply_mlp(lp["ctx_g"], c_V, act=jax.nn.relu, final_act=jax.nn.sigmoid)
        h_V = h_V * gate[batch_id]
    logits = _apply_lin(params["readout"], h_V)
    return jax.nn.log_softmax(logits, axis=-1)


# speedup vs baseline: 3.4511x; 1.1938x over previous
import functools

import jax
import jax.numpy as jnp
from jax import lax
from jax.experimental import pallas as pl
from jax.experimental.pallas import tpu as pltpu
from jax.experimental.pallas import tpu_sc as plsc

N_NODES = 10000
N_EDGES = 320000
HIDDEN = 128
HEADS = 4
N_GRAPHS = 8

_NC = 2
_NS = 16
_NW = _NC * _NS

_BE = 3200
_NBLK = N_EDGES // _BE


def _sc_gather(table, idx, chunk=80):
    E = idx.shape[0]
    D = table.shape[1]
    per_w = E // _NW
    assert per_w * _NW == E and per_w % chunk == 0
    nch = per_w // chunk
    mesh = plsc.VectorSubcoreMesh(
        core_axis_name="c", subcore_axis_name="s", num_cores=_NC, num_subcores=_NS
    )

    @functools.partial(
        pl.kernel,
        mesh=mesh,
        out_type=jax.ShapeDtypeStruct((E, D), jnp.float32),
        scratch_types=[
            pltpu.VMEM((chunk,), jnp.int32),
            pltpu.VMEM((chunk, D), jnp.float32),
            pltpu.SemaphoreType.DMA,
        ],
    )
    def k(table_hbm, idx_hbm, out_hbm, idx_v, rows_v, sem):
        wid = lax.axis_index("s") * _NC + lax.axis_index("c")
        base = wid * per_w

        def body(j, carry):
            off = base + j * chunk
            pltpu.sync_copy(idx_hbm.at[pl.ds(off, chunk)], idx_v)
            pltpu.async_copy(table_hbm.at[idx_v], rows_v, sem).wait()
            pltpu.sync_copy(rows_v, out_hbm.at[pl.ds(off, chunk)])
            return carry

        lax.fori_loop(0, nch, body, 0)

    return k(table, idx)


def _sc_scatter_add(vals, idx, n, chunk=80):
    E, D = vals.shape
    per_w = E // _NW
    assert per_w * _NW == E and per_w % chunk == 0
    nch = per_w // chunk
    n_pad = ((n + 8 * _NS - 1) // (8 * _NS)) * (8 * _NS)
    rows_per_sub = n_pad // _NS
    mesh = plsc.VectorSubcoreMesh(
        core_axis_name="c", subcore_axis_name="s", num_cores=_NC, num_subcores=_NS
    )

    @functools.partial(
        pl.kernel,
        mesh=mesh,
        out_type=jax.ShapeDtypeStruct((_NC, n_pad, D), jnp.float32),
        scratch_types=[
            pltpu.VMEM((chunk,), jnp.int32),
            pltpu.VMEM((chunk, D), jnp.float32),
            pltpu.VMEM_SHARED((n_pad, D), jnp.float32),
            pltpu.SemaphoreType.DMA,
        ],
    )
    def k(zeros_hbm, vals_hbm, idx_hbm, out_hbm, idx_v, vals_v, table_sh, sem):
        cid = lax.axis_index("c")
        sid = lax.axis_index("s")
        wid = sid * _NC + cid
        srow = sid * rows_per_sub
        pltpu.sync_copy(
            zeros_hbm.at[pl.ds(srow, rows_per_sub)],
            table_sh.at[pl.ds(srow, rows_per_sub)],
        )
        plsc.subcore_barrier()
        base = wid * per_w

        def body(j, carry):
            off = base + j * chunk
            pltpu.sync_copy(idx_hbm.at[pl.ds(off, chunk)], idx_v)
            pltpu.sync_copy(vals_hbm.at[pl.ds(off, chunk)], vals_v)
            pltpu.sync_copy(vals_v, table_sh.at[idx_v], add=True)
            return carry

        lax.fori_loop(0, nch, body, 0)
        plsc.subcore_barrier()
        pltpu.sync_copy(
            table_sh.at[pl.ds(srow, rows_per_sub)],
            out_hbm.at[cid, pl.ds(srow, rows_per_sub)],
        )

    return k(jnp.zeros((n_pad, D), jnp.float32), vals, idx)


def _fold_bn(stats_ref, g_ref, b_ref):
    inv_n = 1.0 / N_EDGES
    mu = stats_ref[0:1, :] * inv_n
    var = stats_ref[1:2, :] * inv_n - mu * mu
    sc = g_ref[...] * lax.rsqrt(var + 1e-5)
    sh = b_ref[...] - mu * sc
    return sc, sh


def _edge_blk(i):
    return (i, 0)


def _rep0(i):
    return (0, 0)


def _wspec(shape):
    return pl.BlockSpec(shape, _rep0)


def _edge_attention(xe, g1, g2, lp, emb, stats, bn_g, bn_b, first):
    A1, A2, A3 = lp["att_bias"]
    V1, V2, V3 = lp["w_v"]
    A1we = A1["w"][128:256]
    V1we = V1["w"][0:128]
    a3w8 = jnp.pad(A3["w"], ((0, 0), (0, 4)))
    a3b8 = jnp.pad(A3["b"], (0, 4)).reshape(1, 8)

    def body(xe_ref, g1_ref, g2_ref, a1we, a1b, a2w, a2b, a3w, a3b,
             v1we, v1b, v2w, v2b, v3w, v3b, embw, embb, stats_ref, gref, bref,
             msg_ref, ewf_ref):
        if first:
            wa = embw[...] @ a1we[...]
            ba = embb[...] @ a1we[...] + a1b[...]
            wv = embw[...] @ v1we[...]
            bv = embb[...] @ v1we[...] + v1b[...]
        else:
            sc, sh = _fold_bn(stats_ref, gref, bref)
            wa = a1we[...] * sc.reshape(128, 1)
            ba = sh @ a1we[...] + a1b[...]
            wv = v1we[...] * sc.reshape(128, 1)
            bv = sh @ v1we[...] + v1b[...]
        xe_b = xe_ref[...]
        t1 = jax.nn.gelu(xe_b @ wa + g1_ref[:, 0:128] + g2_ref[...] + ba)
        t2 = jax.nn.gelu(t1 @ a2w[...] + a2b[...])
        w8 = t2 @ a3w[...] + a3b[...]
        ew8 = jnp.exp(w8)
        li = lax.broadcasted_iota(jnp.int32, (8, 128), 1) // 32
        ri = lax.broadcasted_iota(jnp.int32, (8, 128), 0)
        rep = jnp.where((li == ri) & (ri < HEADS), 1.0, 0.0)
        ewf = ew8 @ rep
        u1 = jax.nn.gelu(xe_b @ wv + g1_ref[:, 128:256] + bv)
        u2 = jax.nn.gelu(u1 @ v2w[...] + v2b[...])
        vm = u2 @ v3w[...] + v3b[...]
        ewf_ref[...] = ewf
        msg_ref[...] = ewf * vm

    return pl.pallas_call(
        body,
        grid=(_NBLK,),
        in_specs=[
            pl.BlockSpec((_BE, 128), _edge_blk),
            pl.BlockSpec((_BE, 256), _edge_blk),
            pl.BlockSpec((_BE, 128), _edge_blk),
            _wspec((128, 128)), _wspec((1, 128)),
            _wspec((128, 128)), _wspec((1, 128)),
            _wspec((128, 8)), _wspec((1, 8)),
            _wspec((128, 128)), _wspec((1, 128)),
            _wspec((128, 128)), _wspec((1, 128)),
            _wspec((128, 128)), _wspec((1, 128)),
            _wspec((128, 128)), _wspec((1, 128)),
            _wspec((2, 128)), _wspec((1, 128)), _wspec((1, 128)),
        ],
        out_specs=[
            pl.BlockSpec((_BE, 128), _edge_blk),
            pl.BlockSpec((_BE, 128), _edge_blk),
        ],
        out_shape=[
            jax.ShapeDtypeStruct((N_EDGES, 128), jnp.float32),
            jax.ShapeDtypeStruct((N_EDGES, 128), jnp.float32),
        ],
    )(xe, g1, g2,
      A1we, A1["b"].reshape(1, 128),
      A2["w"], A2["b"].reshape(1, 128),
      a3w8, a3b8,
      V1we, V1["b"].reshape(1, 128),
      V2["w"], V2["b"].reshape(1, 128),
      V3["w"], V3["b"].reshape(1, 128),
      emb["w"], emb["b"].reshape(1, 128),
      stats, bn_g, bn_b)


def _edge_update(xe, g3, g4, lp, emb, stats, bn_g, bn_b, first):
    W11, W12, W13 = lp["w11"], lp["w12"], lp["w13"]
    W11e = W11["w"][128:256]

    def body(xe_ref, g3_ref, g4_ref, w11e, b11, w12w, b12, w13w, b13,
             embw, embb, stats_ref, gref, bref, x_ref, st_ref):
        xe_b = xe_ref[...]
        if first:
            we = embw[...] @ w11e[...]
            be = embb[...] @ w11e[...] + b11[...]
            he = xe_b @ embw[...] + embb[...]
        else:
            sc, sh = _fold_bn(stats_ref, gref, bref)
            we = w11e[...] * sc.reshape(128, 1)
            be = sh @ w11e[...] + b11[...]
            he = xe_b * sc + sh
        q1 = jax.nn.gelu(xe_b @ we + g3_ref[...] + g4_ref[...] + be)
        q2 = jax.nn.gelu(q1 @ w12w[...] + b12[...])
        m = q2 @ w13w[...] + b13[...]
        xn = he + m
        x_ref[...] = xn
        s = jnp.sum(xn, 0, keepdims=True)
        q = jnp.sum(xn * xn, 0, keepdims=True)
        i = pl.program_id(0)

        @pl.when(i == 0)
        def _():
            st_ref[0:1, :] = s
            st_ref[1:2, :] = q

        @pl.when(i != 0)
        def _():
            st_ref[0:1, :] = st_ref[0:1, :] + s
            st_ref[1:2, :] = st_ref[1:2, :] + q

    return pl.pallas_call(
        body,
        grid=(_NBLK,),
        in_specs=[
            pl.BlockSpec((_BE, 128), _edge_blk),
            pl.BlockSpec((_BE, 128), _edge_blk),
            pl.BlockSpec((_BE, 128), _edge_blk),
            _wspec((128, 128)), _wspec((1, 128)),
            _wspec((128, 128)), _wspec((1, 128)),
            _wspec((128, 128)), _wspec((1, 128)),
            _wspec((128, 128)), _wspec((1, 128)),
            _wspec((2, 128)), _wspec((1, 128)), _wspec((1, 128)),
        ],
        out_specs=[
            pl.BlockSpec((_BE, 128), _edge_blk),
            pl.BlockSpec((2, 128), _rep0),
        ],
        out_shape=[
            jax.ShapeDtypeStruct((N_EDGES, 128), jnp.float32),
            jax.ShapeDtypeStruct((2, 128), jnp.float32),
        ],
    )(xe, g3, g4,
      W11e, W11["b"].reshape(1, 128),
      W12["w"], W12["b"].reshape(1, 128),
      W13["w"], W13["b"].reshape(1, 128),
      emb["w"], emb["b"].reshape(1, 128),
      stats, bn_g, bn_b)


def _node_embed_proj(h_V, params):
    lp = params["layers"][0]
    emb = params["node_embed"]
    wsA = lp["att_bias"][0]["w"][256:384]
    wdA = lp["att_bias"][0]["w"][0:128]
    wsV = lp["w_v"][0]["w"][128:256]

    def body(hv_ref, embw, embb, wsa, wda, wsv, hv0_ref, g1_ref, g2_ref):
        hv0 = hv_ref[...] @ embw[...] + embb[...]
        hv0_ref[...] = hv0
        g1_ref[:, 0:128] = hv0 @ wsa[...]
        g1_ref[:, 128:256] = hv0 @ wsv[...]
        g2_ref[...] = hv0 @ wda[...]

    return pl.pallas_call(
        body,
        grid=(1,),
        in_specs=[
            pl.BlockSpec((N_NODES, 128), _rep0),
            _wspec((128, 128)), _wspec((1, 128)),
            _wspec((128, 128)), _wspec((128, 128)), _wspec((128, 128)),
        ],
        out_specs=[
            pl.BlockSpec((N_NODES, 128), _rep0),
            pl.BlockSpec((N_NODES, 256), _rep0),
            pl.BlockSpec((N_NODES, 128), _rep0),
        ],
        out_shape=[
            jax.ShapeDtypeStruct((N_NODES, 128), jnp.float32),
            jax.ShapeDtypeStruct((N_NODES, 256), jnp.float32),
            jax.ShapeDtypeStruct((N_NODES, 128), jnp.float32),
        ],
    )(h_V, emb["w"], emb["b"].reshape(1, 128), wsA, wdA, wsV)


def _node_update1(p, d, h_V, lp):
    n_pad = p.shape[1]

    def body(p_ref, d_ref, hv_ref, wow, wob, g0, b0, x1_ref):
        agg = (p_ref[0, 0:N_NODES, :] + p_ref[1, 0:N_NODES, :]) / (
            d_ref[0, 0:N_NODES, :] + d_ref[1, 0:N_NODES, :] + 1e-9)
        y = hv_ref[...] + agg @ wow[...] + wob[...]
        mu = jnp.mean(y, 0, keepdims=True)
        var = jnp.mean(y * y, 0, keepdims=True) - mu * mu
        x1_ref[...] = (y - mu) * lax.rsqrt(var + 1e-5) * g0[...] + b0[...]

    return pl.pallas_call(
        body,
        grid=(1,),
        in_specs=[
            pl.BlockSpec((_NC, n_pad, 128), lambda i: (0, 0, 0)),
            pl.BlockSpec((_NC, n_pad, 128), lambda i: (0, 0, 0)),
            pl.BlockSpec((N_NODES, 128), _rep0),
            _wspec((128, 128)), _wspec((1, 128)),
            _wspec((1, 128)), _wspec((1, 128)),
        ],
        out_specs=pl.BlockSpec((N_NODES, 128), _rep0),
        out_shape=jax.ShapeDtypeStruct((N_NODES, 128), jnp.float32),
    )(p, d, h_V,
      lp["w_o"]["w"], lp["w_o"]["b"].reshape(1, 128),
      lp["bn0"]["g"].reshape(1, 128), lp["bn0"]["b"].reshape(1, 128))


def _node_update2(x1, lp):
    D1, D2 = lp["dense"]
    CH = 2000

    def body(x1_ref, d1w, d1b, d2w, d2b, g1, b1, v2_ref):
        for c in range(N_NODES // CH):
            xb = x1_ref[pl.ds(c * CH, CH), :]
            mid = jax.nn.gelu(xb @ d1w[...] + d1b[...])
            v2_ref[pl.ds(c * CH, CH), :] = xb + mid @ d2w[...] + d2b[...]
        y = v2_ref[...]
        mu = jnp.mean(y, 0, keepdims=True)
        var = jnp.mean(y * y, 0, keepdims=True) - mu * mu
        v2_ref[...] = (y - mu) * lax.rsqrt(var + 1e-5) * g1[...] + b1[...]

    return pl.pallas_call(
        body,
        grid=(1,),
        in_specs=[
            pl.BlockSpec((N_NODES, 128), _rep0),
            _wspec((128, 512)), _wspec((1, 512)),
            _wspec((512, 128)), _wspec((1, 128)),
            _wspec((1, 128)), _wspec((1, 128)),
        ],
        out_specs=pl.BlockSpec((N_NODES, 128), _rep0),
        out_shape=jax.ShapeDtypeStruct((N_NODES, 128), jnp.float32),
    )(x1, D1["w"], D1["b"].reshape(1, 512), D2["w"], D2["b"].reshape(1, 128),
      lp["bn1"]["g"].reshape(1, 128), lp["bn1"]["b"].reshape(1, 128))


def _node_update3(v2, bid2, lp, next_lp):
    C1, C2, C3 = lp["ctx_g"]
    wsu = lp["w11"]["w"][0:128]
    wdu = lp["w11"]["w"][256:384]
    last = next_lp is None
    if not last:
        wsA = next_lp["att_bias"][0]["w"][256:384]
        wdA = next_lp["att_bias"][0]["w"][0:128]
        wsV = next_lp["w_v"][0]["w"][128:256]
    else:
        wsA = wdA = wsV = jnp.zeros((128, 128), jnp.float32)

    def body(v2_ref, bid_ref, c1w, c1b, c2w, c2b, c3w, c3b,
             wsu_, wdu_, wsa, wda, wsv,
             hg_ref, pu_ref, pd_ref, g1_ref, g2_ref):
        v2b = v2_ref[...]
        oh = jnp.where(
            bid_ref[...] == lax.broadcasted_iota(jnp.int32, (N_NODES, 8), 1),
            1.0, 0.0)
        csum = lax.dot_general(oh, v2b, (((0,), (0,)), ((), ())))
        ones = jnp.full((N_NODES, 1), 1.0, jnp.float32)
        cnt = lax.dot_general(oh, ones, (((0,), (0,)), ((), ())))
        c_V = csum / (cnt + 1e-9)
        gm1 = jax.nn.relu(c_V @ c1w[...] + c1b[...])
        gm2 = jax.nn.relu(gm1 @ c2w[...] + c2b[...])
        gate = jax.nn.sigmoid(gm2 @ c3w[...] + c3b[...])
        hg = v2b * (oh @ gate)
        hg_ref[...] = hg
        pu_ref[...] = v2b @ wsu_[...]
        pd_ref[...] = v2b @ wdu_[...]
        if not last:
            g1_ref[:, 0:128] = hg @ wsa[...]
            g1_ref[:, 128:256] = hg @ wsv[...]
            g2_ref[...] = hg @ wda[...]
        else:
            g1_ref[...] = jnp.zeros((N_NODES, 256), jnp.float32)
            g2_ref[...] = jnp.zeros((N_NODES, 128), jnp.float32)

    return pl.pallas_call(
        body,
        grid=(1,),
        in_specs=[
            pl.BlockSpec((N_NODES, 128), _rep0),
            pl.BlockSpec((N_NODES, 1), _rep0),
            _wspec((128, 128)), _wspec((1, 128)),
            _wspec((128, 128)), _wspec((1, 128)),
            _wspec((128, 128)), _wspec((1, 128)),
            _wspec((128, 128)), _wspec((128, 128)),
            _wspec((128, 128)), _wspec((128, 128)), _wspec((128, 128)),
        ],
        out_specs=[
            pl.BlockSpec((N_NODES, 128), _rep0),
            pl.BlockSpec((N_NODES, 128), _rep0),
            pl.BlockSpec((N_NODES, 128), _rep0),
            pl.BlockSpec((N_NODES, 256), _rep0),
            pl.BlockSpec((N_NODES, 128), _rep0),
        ],
        out_shape=[
            jax.ShapeDtypeStruct((N_NODES, 128), jnp.float32),
            jax.ShapeDtypeStruct((N_NODES, 128), jnp.float32),
            jax.ShapeDtypeStruct((N_NODES, 128), jnp.float32),
            jax.ShapeDtypeStruct((N_NODES, 256), jnp.float32),
            jax.ShapeDtypeStruct((N_NODES, 128), jnp.float32),
        ],
    )(v2, bid2,
      C1["w"], C1["b"].reshape(1, 128),
      C2["w"], C2["b"].reshape(1, 128),
      C3["w"], C3["b"].reshape(1, 128),
      wsu, wdu, wsA, wdA, wsV)


def _readout(hg, params):
    V = params["readout"]["b"].shape[0]

    def body(hg_ref, wr, br, out_ref):
        z = hg_ref[...] @ wr[...] + br[...]
        zmax = jnp.max(z, 1, keepdims=True)
        zc = z - zmax
        lse = jnp.log(jnp.sum(jnp.exp(zc), 1, keepdims=True))
        out_ref[...] = zc - lse

    return pl.pallas_call(
        body,
        grid=(1,),
        in_specs=[
            pl.BlockSpec((N_NODES, 128), _rep0),
            _wspec((128, V)), _wspec((1, V)),
        ],
        out_specs=pl.BlockSpec((N_NODES, V), _rep0),
        out_shape=jax.ShapeDtypeStruct((N_NODES, V), jnp.float32),
    )(hg, params["readout"]["w"], params["readout"]["b"].reshape(1, V))


def kernel(h_V, h_E, params, edge_idx, batch_id):
    src = edge_idx[0]
    dst = edge_idx[1]
    bid2 = batch_id.reshape(N_NODES, 1)
    layers = params["layers"]
    n_emb = params["node_embed"]
    e_emb = params["edge_embed"]

    hv0, g1_tab, g2_tab = _node_embed_proj(h_V, params)
    hv = hv0
    xe = h_E
    stats = jnp.zeros((2, 128), jnp.float32)
    ones128 = jnp.ones((1, 128), jnp.float32)
    zeros128 = jnp.zeros((1, 128), jnp.float32)
    bn_g, bn_b = ones128, zeros128

    for li, lp in enumerate(layers):
        first = li == 0
        last = li == len(layers) - 1
        g1 = _sc_gather(g1_tab, src)
        g2 = _sc_gather(g2_tab, dst)
        msgu, ewf = _edge_attention(xe, g1, g2, lp, e_emb, stats, bn_g, bn_b, first)
        p = _sc_scatter_add(msgu, dst, N_NODES)
        d = _sc_scatter_add(ewf, dst, N_NODES)
        x1 = _node_update1(p, d, hv, lp)
        v2 = _node_update2(x1, lp)
        hg, pu, pd, g1n, g2n = _node_update3(
            v2, bid2, lp, None if last else layers[li + 1])
        if not last:
            g3 = _sc_gather(pu, src)
            g4 = _sc_gather(pd, dst)
            xe, stats = _edge_update(xe, g3, g4, lp, e_emb, stats, bn_g, bn_b, first)
            bn_g = lp["bn_e"]["g"].reshape(1, 128)
            bn_b = lp["bn_e"]["b"].reshape(1, 128)
            g1_tab, g2_tab = g1n, g2n
        hv = hg
    return _readout(hv, params)


# speedup vs baseline: 4.7853x; 1.3866x over previous
import functools

import jax
import jax.numpy as jnp
from jax import lax
from jax.experimental import pallas as pl
from jax.experimental.pallas import tpu as pltpu
from jax.experimental.pallas import tpu_sc as plsc

N_NODES = 10000
N_EDGES = 320000
HIDDEN = 128
HEADS = 4
N_GRAPHS = 8

_NC = 2
_NS = 16
_NW = _NC * _NS

_BE = 3200
_NBLK = N_EDGES // _BE


def _sc_gather(table, idx3, chunk=80, k=5):
    D = table.shape[1]
    NW_, nch, C = idx3.shape
    assert NW_ == _NW and C == chunk and nch % k == 0
    per_w = nch * C
    E = _NW * per_w
    mesh = plsc.VectorSubcoreMesh(
        core_axis_name="c", subcore_axis_name="s", num_cores=_NC, num_subcores=_NS
    )

    @functools.partial(
        pl.kernel,
        mesh=mesh,
        out_type=jax.ShapeDtypeStruct((E, D), jnp.float32),
        scratch_types=[
            pltpu.VMEM((nch, C), jnp.int32),
            pltpu.VMEM((k, C, D), jnp.float32),
            pltpu.SemaphoreType.DMA,
            pltpu.SemaphoreType.DMA,
        ],
    )
    def kk(table_hbm, idx_hbm, out_hbm, idx_v, rows_v, sem_g, sem_w):
        wid = lax.axis_index("s") * _NC + lax.axis_index("c")
        base = wid * per_w
        pltpu.sync_copy(idx_hbm.at[wid], idx_v)

        def sup(sj, carry):
            j0 = sj * k
            gds = [
                pltpu.async_copy(
                    table_hbm.at[idx_v.at[j0 + b]], rows_v.at[b], sem_g)
                for b in range(k)
            ]
            for gd in gds:
                gd.wait()
            wds = [
                pltpu.async_copy(
                    rows_v.at[b],
                    out_hbm.at[pl.ds(base + (j0 + b) * C, C)], sem_w)
                for b in range(k)
            ]
            for wd in wds:
                wd.wait()
            return carry

        lax.fori_loop(0, nch // k, sup, 0)

    return kk(table, idx3)


def _sc_scatter_add(vals, idx3, n, chunk=40, k=5):
    E, D = vals.shape
    NW_, nsup, k8, C = idx3.shape
    assert NW_ == _NW and C == chunk and k8 == 8
    per_w = nsup * k * C
    assert per_w * _NW == E
    n_pad = ((n + 8 * _NS - 1) // (8 * _NS)) * (8 * _NS)
    rows_per_sub = n_pad // _NS
    mesh = plsc.VectorSubcoreMesh(
        core_axis_name="c", subcore_axis_name="s", num_cores=_NC, num_subcores=_NS
    )

    @functools.partial(
        pl.kernel,
        mesh=mesh,
        out_type=jax.ShapeDtypeStruct((_NC, n_pad, D), jnp.float32),
        scratch_types=[
            pltpu.VMEM((8, C), jnp.int32),
            pltpu.VMEM((k, C, D), jnp.float32),
            pltpu.VMEM_SHARED((n_pad, D), jnp.float32),
            pltpu.SemaphoreType.DMA,
            pltpu.SemaphoreType.DMA,
        ],
    )
    def kk(zeros_hbm, vals_hbm, idx_hbm, out_hbm, idx_v, vals_v, table_sh,
           sem_l, sem_s):
        cid = lax.axis_index("c")
        sid = lax.axis_index("s")
        wid = sid * _NC + cid
        srow = sid * rows_per_sub
        pltpu.sync_copy(
            zeros_hbm.at[pl.ds(srow, rows_per_sub)],
            table_sh.at[pl.ds(srow, rows_per_sub)],
        )
        plsc.subcore_barrier()
        base = wid * per_w

        def sup(sj, carry):
            j0 = sj * k
            pltpu.sync_copy(idx_hbm.at[wid, sj], idx_v)
            lds = [
                pltpu.async_copy(
                    vals_hbm.at[pl.ds(base + (j0 + b) * C, C)],
                    vals_v.at[b], sem_l)
                for b in range(k)
            ]
            for ld in lds:
                ld.wait()
            sds = [
                pltpu.async_copy(
                    vals_v.at[b], table_sh.at[idx_v.at[b]],
                    sem_s, add=True)
                for b in range(k)
            ]
            for sd in sds:
                sd.wait()
            return carry

        lax.fori_loop(0, nsup, sup, 0)
        plsc.subcore_barrier()
        pltpu.sync_copy(
            table_sh.at[pl.ds(srow, rows_per_sub)],
            out_hbm.at[cid, pl.ds(srow, rows_per_sub)],
        )

    return kk(jnp.zeros((n_pad, D), jnp.float32), vals, idx3)


def _fold_bn(stats_ref, g_ref, b_ref):
    inv_n = 1.0 / N_EDGES
    mu = stats_ref[0:1, :] * inv_n
    var = stats_ref[1:2, :] * inv_n - mu * mu
    sc = g_ref[...] * lax.rsqrt(var + 1e-5)
    sh = b_ref[...] - mu * sc
    return sc, sh


def _edge_blk(i):
    return (i, 0)


def _rep0(i):
    return (0, 0)


def _wspec(shape):
    return pl.BlockSpec(shape, _rep0)


def _edge_attention(xe, g1, g2, lp, emb, stats, bn_g, bn_b, first):
    A1, A2, A3 = lp["att_bias"]
    V1, V2, V3 = lp["w_v"]
    A1we = A1["w"][128:256]
    V1we = V1["w"][0:128]
    a3w8 = jnp.pad(A3["w"], ((0, 0), (0, 4)))
    a3b8 = jnp.pad(A3["b"], (0, 4)).reshape(1, 8)

    def body(xe_ref, g1_ref, g2_ref, a1we, a1b, a2w, a2b, a3w, a3b,
             v1we, v1b, v2w, v2b, v3w, v3b, embw, embb, stats_ref, gref, bref,
             msg_ref, ewf_ref):
        if first:
            wa = embw[...] @ a1we[...]
            ba = embb[...] @ a1we[...] + a1b[...]
            wv = embw[...] @ v1we[...]
            bv = embb[...] @ v1we[...] + v1b[...]
        else:
            sc, sh = _fold_bn(stats_ref, gref, bref)
            wa = a1we[...] * sc.reshape(128, 1)
            ba = sh @ a1we[...] + a1b[...]
            wv = v1we[...] * sc.reshape(128, 1)
            bv = sh @ v1we[...] + v1b[...]
        xe_b = xe_ref[...]
        t1 = jax.nn.gelu(xe_b @ wa + g1_ref[:, 0:128] + g2_ref[...] + ba)
        t2 = jax.nn.gelu(t1 @ a2w[...] + a2b[...])
        w8 = t2 @ a3w[...] + a3b[...]
        ew8 = jnp.exp(w8)
        li = lax.broadcasted_iota(jnp.int32, (8, 128), 1) // 32
        ri = lax.broadcasted_iota(jnp.int32, (8, 128), 0)
        rep = jnp.where((li == ri) & (ri < HEADS), 1.0, 0.0)
        ewf = ew8 @ rep
        u1 = jax.nn.gelu(xe_b @ wv + g1_ref[:, 128:256] + bv)
        u2 = jax.nn.gelu(u1 @ v2w[...] + v2b[...])
        vm = u2 @ v3w[...] + v3b[...]
        ewf_ref[...] = ewf
        msg_ref[...] = ewf * vm

    return pl.pallas_call(
        body,
        grid=(_NBLK,),
        in_specs=[
            pl.BlockSpec((_BE, 128), _edge_blk),
            pl.BlockSpec((_BE, 256), _edge_blk),
            pl.BlockSpec((_BE, 128), _edge_blk),
            _wspec((128, 128)), _wspec((1, 128)),
            _wspec((128, 128)), _wspec((1, 128)),
            _wspec((128, 8)), _wspec((1, 8)),
            _wspec((128, 128)), _wspec((1, 128)),
            _wspec((128, 128)), _wspec((1, 128)),
            _wspec((128, 128)), _wspec((1, 128)),
            _wspec((128, 128)), _wspec((1, 128)),
            _wspec((2, 128)), _wspec((1, 128)), _wspec((1, 128)),
        ],
        out_specs=[
            pl.BlockSpec((_BE, 128), _edge_blk),
            pl.BlockSpec((_BE, 128), _edge_blk),
        ],
        out_shape=[
            jax.ShapeDtypeStruct((N_EDGES, 128), jnp.float32),
            jax.ShapeDtypeStruct((N_EDGES, 128), jnp.float32),
        ],
    )(xe, g1, g2,
      A1we, A1["b"].reshape(1, 128),
      A2["w"], A2["b"].reshape(1, 128),
      a3w8, a3b8,
      V1we, V1["b"].reshape(1, 128),
      V2["w"], V2["b"].reshape(1, 128),
      V3["w"], V3["b"].reshape(1, 128),
      emb["w"], emb["b"].reshape(1, 128),
      stats, bn_g, bn_b)


def _edge_update(xe, g3, g4, lp, emb, stats, bn_g, bn_b, first):
    W11, W12, W13 = lp["w11"], lp["w12"], lp["w13"]
    W11e = W11["w"][128:256]

    def body(xe_ref, g3_ref, g4_ref, w11e, b11, w12w, b12, w13w, b13,
             embw, embb, stats_ref, gref, bref, x_ref, st_ref):
        xe_b = xe_ref[...]
        if first:
            we = embw[...] @ w11e[...]
            be = embb[...] @ w11e[...] + b11[...]
            he = xe_b @ embw[...] + embb[...]
        else:
            sc, sh = _fold_bn(stats_ref, gref, bref)
            we = w11e[...] * sc.reshape(128, 1)
            be = sh @ w11e[...] + b11[...]
            he = xe_b * sc + sh
        q1 = jax.nn.gelu(xe_b @ we + g3_ref[...] + g4_ref[...] + be)
        q2 = jax.nn.gelu(q1 @ w12w[...] + b12[...])
        m = q2 @ w13w[...] + b13[...]
        xn = he + m
        x_ref[...] = xn
        s = jnp.sum(xn, 0, keepdims=True)
        q = jnp.sum(xn * xn, 0, keepdims=True)
        i = pl.program_id(0)

        @pl.when(i == 0)
        def _():
            st_ref[0:1, :] = s
            st_ref[1:2, :] = q

        @pl.when(i != 0)
        def _():
            st_ref[0:1, :] = st_ref[0:1, :] + s
            st_ref[1:2, :] = st_ref[1:2, :] + q

    return pl.pallas_call(
        body,
        grid=(_NBLK,),
        in_specs=[
            pl.BlockSpec((_BE, 128), _edge_blk),
            pl.BlockSpec((_BE, 128), _edge_blk),
            pl.BlockSpec((_BE, 128), _edge_blk),
            _wspec((128, 128)), _wspec((1, 128)),
            _wspec((128, 128)), _wspec((1, 128)),
            _wspec((128, 128)), _wspec((1, 128)),
            _wspec((128, 128)), _wspec((1, 128)),
            _wspec((2, 128)), _wspec((1, 128)), _wspec((1, 128)),
        ],
        out_specs=[
            pl.BlockSpec((_BE, 128), _edge_blk),
            pl.BlockSpec((2, 128), _rep0),
        ],
        out_shape=[
            jax.ShapeDtypeStruct((N_EDGES, 128), jnp.float32),
            jax.ShapeDtypeStruct((2, 128), jnp.float32),
        ],
    )(xe, g3, g4,
      W11e, W11["b"].reshape(1, 128),
      W12["w"], W12["b"].reshape(1, 128),
      W13["w"], W13["b"].reshape(1, 128),
      emb["w"], emb["b"].reshape(1, 128),
      stats, bn_g, bn_b)


def _node_embed_proj(h_V, params):
    lp = params["layers"][0]
    emb = params["node_embed"]
    wsA = lp["att_bias"][0]["w"][256:384]
    wdA = lp["att_bias"][0]["w"][0:128]
    wsV = lp["w_v"][0]["w"][128:256]

    def body(hv_ref, embw, embb, wsa, wda, wsv, hv0_ref, g1_ref, g2_ref):
        hv0 = hv_ref[...] @ embw[...] + embb[...]
        hv0_ref[...] = hv0
        g1_ref[:, 0:128] = hv0 @ wsa[...]
        g1_ref[:, 128:256] = hv0 @ wsv[...]
        g2_ref[...] = hv0 @ wda[...]

    return pl.pallas_call(
        body,
        grid=(1,),
        in_specs=[
            pl.BlockSpec((N_NODES, 128), _rep0),
            _wspec((128, 128)), _wspec((1, 128)),
            _wspec((128, 128)), _wspec((128, 128)), _wspec((128, 128)),
        ],
        out_specs=[
            pl.BlockSpec((N_NODES, 128), _rep0),
            pl.BlockSpec((N_NODES, 256), _rep0),
            pl.BlockSpec((N_NODES, 128), _rep0),
        ],
        out_shape=[
            jax.ShapeDtypeStruct((N_NODES, 128), jnp.float32),
            jax.ShapeDtypeStruct((N_NODES, 256), jnp.float32),
            jax.ShapeDtypeStruct((N_NODES, 128), jnp.float32),
        ],
    )(h_V, emb["w"], emb["b"].reshape(1, 128), wsA, wdA, wsV)


def _node_update1(p, d, h_V, lp):
    n_pad = p.shape[1]

    def body(p_ref, d_ref, hv_ref, wow, wob, g0, b0, x1_ref):
        agg = (p_ref[0, 0:N_NODES, :] + p_ref[1, 0:N_NODES, :]) / (
            d_ref[0, 0:N_NODES, :] + d_ref[1, 0:N_NODES, :] + 1e-9)
        y = hv_ref[...] + agg @ wow[...] + wob[...]
        mu = jnp.mean(y, 0, keepdims=True)
        var = jnp.mean(y * y, 0, keepdims=True) - mu * mu
        x1_ref[...] = (y - mu) * lax.rsqrt(var + 1e-5) * g0[...] + b0[...]

    return pl.pallas_call(
        body,
        grid=(1,),
        in_specs=[
            pl.BlockSpec((_NC, n_pad, 128), lambda i: (0, 0, 0)),
            pl.BlockSpec((_NC, n_pad, 128), lambda i: (0, 0, 0)),
            pl.BlockSpec((N_NODES, 128), _rep0),
            _wspec((128, 128)), _wspec((1, 128)),
            _wspec((1, 128)), _wspec((1, 128)),
        ],
        out_specs=pl.BlockSpec((N_NODES, 128), _rep0),
        out_shape=jax.ShapeDtypeStruct((N_NODES, 128), jnp.float32),
    )(p, d, h_V,
      lp["w_o"]["w"], lp["w_o"]["b"].reshape(1, 128),
      lp["bn0"]["g"].reshape(1, 128), lp["bn0"]["b"].reshape(1, 128))


def _node_update2(x1, lp):
    D1, D2 = lp["dense"]
    CH = 2000

    def body(x1_ref, d1w, d1b, d2w, d2b, g1, b1, v2_ref):
        for c in range(N_NODES // CH):
            xb = x1_ref[pl.ds(c * CH, CH), :]
            mid = jax.nn.gelu(xb @ d1w[...] + d1b[...])
            v2_ref[pl.ds(c * CH, CH), :] = xb + mid @ d2w[...] + d2b[...]
        y = v2_ref[...]
        mu = jnp.mean(y, 0, keepdims=True)
        var = jnp.mean(y * y, 0, keepdims=True) - mu * mu
        v2_ref[...] = (y - mu) * lax.rsqrt(var + 1e-5) * g1[...] + b1[...]

    return pl.pallas_call(
        body,
        grid=(1,),
        in_specs=[
            pl.BlockSpec((N_NODES, 128), _rep0),
            _wspec((128, 512)), _wspec((1, 512)),
            _wspec((512, 128)), _wspec((1, 128)),
            _wspec((1, 128)), _wspec((1, 128)),
        ],
        out_specs=pl.BlockSpec((N_NODES, 128), _rep0),
        out_shape=jax.ShapeDtypeStruct((N_NODES, 128), jnp.float32),
    )(x1, D1["w"], D1["b"].reshape(1, 512), D2["w"], D2["b"].reshape(1, 128),
      lp["bn1"]["g"].reshape(1, 128), lp["bn1"]["b"].reshape(1, 128))


def _node_update3(v2, bid2, lp, next_lp):
    C1, C2, C3 = lp["ctx_g"]
    wsu = lp["w11"]["w"][0:128]
    wdu = lp["w11"]["w"][256:384]
    last = next_lp is None
    if not last:
        wsA = next_lp["att_bias"][0]["w"][256:384]
        wdA = next_lp["att_bias"][0]["w"][0:128]
        wsV = next_lp["w_v"][0]["w"][128:256]
    else:
        wsA = wdA = wsV = jnp.zeros((128, 128), jnp.float32)

    def body(v2_ref, bid_ref, c1w, c1b, c2w, c2b, c3w, c3b,
             wsu_, wdu_, wsa, wda, wsv,
             hg_ref, pu_ref, pd_ref, g1_ref, g2_ref):
        v2b = v2_ref[...]
        oh = jnp.where(
            bid_ref[...] == lax.broadcasted_iota(jnp.int32, (N_NODES, 8), 1),
            1.0, 0.0)
        csum = lax.dot_general(oh, v2b, (((0,), (0,)), ((), ())))
        ones = jnp.full((N_NODES, 1), 1.0, jnp.float32)
        cnt = lax.dot_general(oh, ones, (((0,), (0,)), ((), ())))
        c_V = csum / (cnt + 1e-9)
        gm1 = jax.nn.relu(c_V @ c1w[...] + c1b[...])
        gm2 = jax.nn.relu(gm1 @ c2w[...] + c2b[...])
        gate = jax.nn.sigmoid(gm2 @ c3w[...] + c3b[...])
        hg = v2b * (oh @ gate)
        hg_ref[...] = hg
        pu_ref[...] = v2b @ wsu_[...]
        pd_ref[...] = v2b @ wdu_[...]
        if not last:
            g1_ref[:, 0:128] = hg @ wsa[...]
            g1_ref[:, 128:256] = hg @ wsv[...]
            g2_ref[...] = hg @ wda[...]
        else:
            g1_ref[...] = jnp.zeros((N_NODES, 256), jnp.float32)
            g2_ref[...] = jnp.zeros((N_NODES, 128), jnp.float32)

    return pl.pallas_call(
        body,
        grid=(1,),
        in_specs=[
            pl.BlockSpec((N_NODES, 128), _rep0),
            pl.BlockSpec((N_NODES, 1), _rep0),
            _wspec((128, 128)), _wspec((1, 128)),
            _wspec((128, 128)), _wspec((1, 128)),
            _wspec((128, 128)), _wspec((1, 128)),
            _wspec((128, 128)), _wspec((128, 128)),
            _wspec((128, 128)), _wspec((128, 128)), _wspec((128, 128)),
        ],
        out_specs=[
            pl.BlockSpec((N_NODES, 128), _rep0),
            pl.BlockSpec((N_NODES, 128), _rep0),
            pl.BlockSpec((N_NODES, 128), _rep0),
            pl.BlockSpec((N_NODES, 256), _rep0),
            pl.BlockSpec((N_NODES, 128), _rep0),
        ],
        out_shape=[
            jax.ShapeDtypeStruct((N_NODES, 128), jnp.float32),
            jax.ShapeDtypeStruct((N_NODES, 128), jnp.float32),
            jax.ShapeDtypeStruct((N_NODES, 128), jnp.float32),
            jax.ShapeDtypeStruct((N_NODES, 256), jnp.float32),
            jax.ShapeDtypeStruct((N_NODES, 128), jnp.float32),
        ],
    )(v2, bid2,
      C1["w"], C1["b"].reshape(1, 128),
      C2["w"], C2["b"].reshape(1, 128),
      C3["w"], C3["b"].reshape(1, 128),
      wsu, wdu, wsA, wdA, wsV)


def _readout(hg, params):
    V = params["readout"]["b"].shape[0]

    def body(hg_ref, wr, br, out_ref):
        z = hg_ref[...] @ wr[...] + br[...]
        zmax = jnp.max(z, 1, keepdims=True)
        zc = z - zmax
        lse = jnp.log(jnp.sum(jnp.exp(zc), 1, keepdims=True))
        out_ref[...] = zc - lse

    return pl.pallas_call(
        body,
        grid=(1,),
        in_specs=[
            pl.BlockSpec((N_NODES, 128), _rep0),
            _wspec((128, V)), _wspec((1, V)),
        ],
        out_specs=pl.BlockSpec((N_NODES, V), _rep0),
        out_shape=jax.ShapeDtypeStruct((N_NODES, V), jnp.float32),
    )(hg, params["readout"]["w"], params["readout"]["b"].reshape(1, V))


def kernel(h_V, h_E, params, edge_idx, batch_id):
    src3 = edge_idx[0].reshape(_NW, -1, 80)
    dst3 = edge_idx[1].reshape(_NW, -1, 80)
    dst3s = jnp.pad(
        edge_idx[1].reshape(_NW, -1, 5, 40), ((0, 0), (0, 0), (0, 3), (0, 0)))
    bid2 = batch_id.reshape(N_NODES, 1)
    layers = params["layers"]
    n_emb = params["node_embed"]
    e_emb = params["edge_embed"]

    hv0, g1_tab, g2_tab = _node_embed_proj(h_V, params)
    hv = hv0
    xe = h_E
    stats = jnp.zeros((2, 128), jnp.float32)
    ones128 = jnp.ones((1, 128), jnp.float32)
    zeros128 = jnp.zeros((1, 128), jnp.float32)
    bn_g, bn_b = ones128, zeros128

    for li, lp in enumerate(layers):
        first = li == 0
        last = li == len(layers) - 1
        g1 = _sc_gather(g1_tab, src3)
        g2 = _sc_gather(g2_tab, dst3)
        msgu, ewf = _edge_attention(xe, g1, g2, lp, e_emb, stats, bn_g, bn_b, first)
        p = _sc_scatter_add(msgu, dst3s, N_NODES)
        d = _sc_scatter_add(ewf, dst3s, N_NODES)
        x1 = _node_update1(p, d, hv, lp)
        v2 = _node_update2(x1, lp)
        hg, pu, pd, g1n, g2n = _node_update3(
            v2, bid2, lp, None if last else layers[li + 1])
        if not last:
            g3 = _sc_gather(pu, src3)
            g4 = _sc_gather(pd, dst3)
            xe, stats = _edge_update(xe, g3, g4, lp, e_emb, stats, bn_g, bn_b, first)
            bn_g = lp["bn_e"]["g"].reshape(1, 128)
            bn_b = lp["bn_e"]["b"].reshape(1, 128)
            g1_tab, g2_tab = g1n, g2n
        hv = hg
    return _readout(hv, params)


# speedup vs baseline: 5.0911x; 1.0639x over previous
import functools

import jax
import jax.numpy as jnp
from jax import lax
from jax.experimental import pallas as pl
from jax.experimental.pallas import tpu as pltpu
from jax.experimental.pallas import tpu_sc as plsc

N_NODES = 10000
N_EDGES = 320000
HIDDEN = 128
HEADS = 4
N_GRAPHS = 8

_NC = 2
_NS = 16
_NW = _NC * _NS

_BE = 3200
_NBLK = N_EDGES // _BE


def _sc_gather(table, idx3, chunk=80, k=5):
    D = table.shape[1]
    NW_, nch, C = idx3.shape
    assert NW_ == _NW and C == chunk and nch % k == 0
    per_w = nch * C
    E = _NW * per_w
    mesh = plsc.VectorSubcoreMesh(
        core_axis_name="c", subcore_axis_name="s", num_cores=_NC, num_subcores=_NS
    )

    @functools.partial(
        pl.kernel,
        mesh=mesh,
        out_type=jax.ShapeDtypeStruct((E, D), jnp.float32),
        scratch_types=[
            pltpu.VMEM((nch, C), jnp.int32),
            pltpu.VMEM((k, C, D), jnp.float32),
            pltpu.SemaphoreType.DMA,
            pltpu.SemaphoreType.DMA,
        ],
    )
    def kk(table_hbm, idx_hbm, out_hbm, idx_v, rows_v, sem_g, sem_w):
        wid = lax.axis_index("s") * _NC + lax.axis_index("c")
        base = wid * per_w
        pltpu.sync_copy(idx_hbm.at[wid], idx_v)

        def sup(sj, carry):
            j0 = sj * k
            gds = [
                pltpu.async_copy(
                    table_hbm.at[idx_v.at[j0 + b]], rows_v.at[b], sem_g)
                for b in range(k)
            ]
            wds = []
            for b in range(k):
                gds[b].wait()
                wds.append(pltpu.async_copy(
                    rows_v.at[b],
                    out_hbm.at[pl.ds(base + (j0 + b) * C, C)], sem_w))
            for wd in wds:
                wd.wait()
            return carry

        lax.fori_loop(0, nch // k, sup, 0)

    return kk(table, idx3)


def _sc_scatter_add2(vals_a, vals_b, idx4, n, chunk=40, k=5):
    E, D = vals_a.shape
    NW_, nsup, k8, C = idx4.shape
    assert NW_ == _NW and C == chunk and k8 == 8
    per_w = nsup * k * C
    assert per_w * _NW == E
    n_pad = ((n + 8 * _NS - 1) // (8 * _NS)) * (8 * _NS)
    rows_per_sub = n_pad // _NS
    mesh = plsc.VectorSubcoreMesh(
        core_axis_name="c", subcore_axis_name="s", num_cores=_NC, num_subcores=_NS
    )

    @functools.partial(
        pl.kernel,
        mesh=mesh,
        out_type=(
            jax.ShapeDtypeStruct((_NC, n_pad, D), jnp.float32),
            jax.ShapeDtypeStruct((_NC, n_pad, D), jnp.float32),
        ),
        scratch_types=[
            pltpu.VMEM((8, C), jnp.int32),
            pltpu.VMEM((k, C, D), jnp.float32),
            pltpu.VMEM_SHARED((n_pad, D), jnp.float32),
            pltpu.SemaphoreType.DMA,
            pltpu.SemaphoreType.DMA,
        ],
    )
    def kk(zeros_hbm, va_hbm, vb_hbm, idx_hbm, outa_hbm, outb_hbm,
           idx_v, vals_v, table_sh, sem_l, sem_s):
        cid = lax.axis_index("c")
        sid = lax.axis_index("s")
        wid = sid * _NC + cid
        srow = sid * rows_per_sub
        base = wid * per_w

        def zero_table():
            pltpu.sync_copy(
                zeros_hbm.at[pl.ds(srow, rows_per_sub)],
                table_sh.at[pl.ds(srow, rows_per_sub)],
            )

        def run(v_hbm, out_hbm):
            def sup(sj, carry):
                j0 = sj * k
                pltpu.sync_copy(idx_hbm.at[wid, sj], idx_v)
                lds = [
                    pltpu.async_copy(
                        v_hbm.at[pl.ds(base + (j0 + b) * C, C)],
                        vals_v.at[b], sem_l)
                    for b in range(k)
                ]
                sds = []
                for b in range(k):
                    lds[b].wait()
                    sds.append(pltpu.async_copy(
                        vals_v.at[b], table_sh.at[idx_v.at[b]],
                        sem_s, add=True))
                for sd in sds:
                    sd.wait()
                return carry

            lax.fori_loop(0, nsup, sup, 0)
            plsc.subcore_barrier()
            pltpu.sync_copy(
                table_sh.at[pl.ds(srow, rows_per_sub)],
                out_hbm.at[cid, pl.ds(srow, rows_per_sub)],
            )

        zero_table()
        plsc.subcore_barrier()
        run(va_hbm, outa_hbm)
        plsc.subcore_barrier()
        zero_table()
        plsc.subcore_barrier()
        run(vb_hbm, outb_hbm)

    return kk(jnp.zeros((n_pad, D), jnp.float32), vals_a, vals_b, idx4)


def _fold_bn(stats_ref, g_ref, b_ref):
    inv_n = 1.0 / N_EDGES
    mu = stats_ref[0:1, :] * inv_n
    var = stats_ref[1:2, :] * inv_n - mu * mu
    sc = g_ref[...] * lax.rsqrt(var + 1e-5)
    sh = b_ref[...] - mu * sc
    return sc, sh


def _edge_blk(i):
    return (i, 0)


def _rep0(i):
    return (0, 0)


def _wspec(shape):
    return pl.BlockSpec(shape, _rep0)


def _edge_attention(xe, g1, g2, lp, emb, stats, bn_g, bn_b, first):
    A1, A2, A3 = lp["att_bias"]
    V1, V2, V3 = lp["w_v"]
    A1we = A1["w"][128:256]
    V1we = V1["w"][0:128]
    a3w8 = jnp.pad(A3["w"], ((0, 0), (0, 4)))
    a3b8 = jnp.pad(A3["b"], (0, 4)).reshape(1, 8)

    def body(xe_ref, g1_ref, g2_ref, a1we, a1b, a2w, a2b, a3w, a3b,
             v1we, v1b, v2w, v2b, v3w, v3b, embw, embb, stats_ref, gref, bref,
             msg_ref, ewf_ref):
        if first:
            wa = embw[...] @ a1we[...]
            ba = embb[...] @ a1we[...] + a1b[...]
            wv = embw[...] @ v1we[...]
            bv = embb[...] @ v1we[...] + v1b[...]
        else:
            sc, sh = _fold_bn(stats_ref, gref, bref)
            wa = a1we[...] * sc.reshape(128, 1)
            ba = sh @ a1we[...] + a1b[...]
            wv = v1we[...] * sc.reshape(128, 1)
            bv = sh @ v1we[...] + v1b[...]
        xe_b = xe_ref[...]
        t1 = jax.nn.gelu(xe_b @ wa + g1_ref[:, 0:128] + g2_ref[...] + ba)
        t2 = jax.nn.gelu(t1 @ a2w[...] + a2b[...])
        w8 = t2 @ a3w[...] + a3b[...]
        ew8 = jnp.exp(w8)
        li = lax.broadcasted_iota(jnp.int32, (8, 128), 1) // 32
        ri = lax.broadcasted_iota(jnp.int32, (8, 128), 0)
        rep = jnp.where((li == ri) & (ri < HEADS), 1.0, 0.0)
        ewf = ew8 @ rep
        u1 = jax.nn.gelu(xe_b @ wv + g1_ref[:, 128:256] + bv)
        u2 = jax.nn.gelu(u1 @ v2w[...] + v2b[...])
        vm = u2 @ v3w[...] + v3b[...]
        ewf_ref[...] = ewf
        msg_ref[...] = ewf * vm

    return pl.pallas_call(
        body,
        grid=(_NBLK,),
        in_specs=[
            pl.BlockSpec((_BE, 128), _edge_blk),
            pl.BlockSpec((_BE, 256), _edge_blk),
            pl.BlockSpec((_BE, 128), _edge_blk),
            _wspec((128, 128)), _wspec((1, 128)),
            _wspec((128, 128)), _wspec((1, 128)),
            _wspec((128, 8)), _wspec((1, 8)),
            _wspec((128, 128)), _wspec((1, 128)),
            _wspec((128, 128)), _wspec((1, 128)),
            _wspec((128, 128)), _wspec((1, 128)),
            _wspec((128, 128)), _wspec((1, 128)),
            _wspec((2, 128)), _wspec((1, 128)), _wspec((1, 128)),
        ],
        out_specs=[
            pl.BlockSpec((_BE, 128), _edge_blk),
            pl.BlockSpec((_BE, 128), _edge_blk),
        ],
        out_shape=[
            jax.ShapeDtypeStruct((N_EDGES, 128), jnp.float32),
            jax.ShapeDtypeStruct((N_EDGES, 128), jnp.float32),
        ],
    )(xe, g1, g2,
      A1we, A1["b"].reshape(1, 128),
      A2["w"], A2["b"].reshape(1, 128),
      a3w8, a3b8,
      V1we, V1["b"].reshape(1, 128),
      V2["w"], V2["b"].reshape(1, 128),
      V3["w"], V3["b"].reshape(1, 128),
      emb["w"], emb["b"].reshape(1, 128),
      stats, bn_g, bn_b)


def _edge_update(xe, g3, g4, lp, emb, stats, bn_g, bn_b, first):
    W11, W12, W13 = lp["w11"], lp["w12"], lp["w13"]
    W11e = W11["w"][128:256]

    def body(xe_ref, g3_ref, g4_ref, w11e, b11, w12w, b12, w13w, b13,
             embw, embb, stats_ref, gref, bref, x_ref, st_ref):
        xe_b = xe_ref[...]
        if first:
            we = embw[...] @ w11e[...]
            be = embb[...] @ w11e[...] + b11[...]
            he = xe_b @ embw[...] + embb[...]
        else:
            sc, sh = _fold_bn(stats_ref, gref, bref)
            we = w11e[...] * sc.reshape(128, 1)
            be = sh @ w11e[...] + b11[...]
            he = xe_b * sc + sh
        q1 = jax.nn.gelu(xe_b @ we + g3_ref[...] + g4_ref[...] + be)
        q2 = jax.nn.gelu(q1 @ w12w[...] + b12[...])
        m = q2 @ w13w[...] + b13[...]
        xn = he + m
        x_ref[...] = xn
        s = jnp.sum(xn, 0, keepdims=True)
        q = jnp.sum(xn * xn, 0, keepdims=True)
        i = pl.program_id(0)

        @pl.when(i == 0)
        def _():
            st_ref[0:1, :] = s
            st_ref[1:2, :] = q

        @pl.when(i != 0)
        def _():
            st_ref[0:1, :] = st_ref[0:1, :] + s
            st_ref[1:2, :] = st_ref[1:2, :] + q

    return pl.pallas_call(
        body,
        grid=(_NBLK,),
        in_specs=[
            pl.BlockSpec((_BE, 128), _edge_blk),
            pl.BlockSpec((_BE, 128), _edge_blk),
            pl.BlockSpec((_BE, 128), _edge_blk),
            _wspec((128, 128)), _wspec((1, 128)),
            _wspec((128, 128)), _wspec((1, 128)),
            _wspec((128, 128)), _wspec((1, 128)),
            _wspec((128, 128)), _wspec((1, 128)),
            _wspec((2, 128)), _wspec((1, 128)), _wspec((1, 128)),
        ],
        out_specs=[
            pl.BlockSpec((_BE, 128), _edge_blk),
            pl.BlockSpec((2, 128), _rep0),
        ],
        out_shape=[
            jax.ShapeDtypeStruct((N_EDGES, 128), jnp.float32),
            jax.ShapeDtypeStruct((2, 128), jnp.float32),
        ],
    )(xe, g3, g4,
      W11e, W11["b"].reshape(1, 128),
      W12["w"], W12["b"].reshape(1, 128),
      W13["w"], W13["b"].reshape(1, 128),
      emb["w"], emb["b"].reshape(1, 128),
      stats, bn_g, bn_b)


def _node_embed_proj(h_V, params):
    lp = params["layers"][0]
    emb = params["node_embed"]
    wsA = lp["att_bias"][0]["w"][256:384]
    wdA = lp["att_bias"][0]["w"][0:128]
    wsV = lp["w_v"][0]["w"][128:256]

    def body(hv_ref, embw, embb, wsa, wda, wsv, hv0_ref, g1_ref, g2_ref):
        hv0 = hv_ref[...] @ embw[...] + embb[...]
        hv0_ref[...] = hv0
        g1_ref[:, 0:128] = hv0 @ wsa[...]
        g1_ref[:, 128:256] = hv0 @ wsv[...]
        g2_ref[...] = hv0 @ wda[...]

    return pl.pallas_call(
        body,
        grid=(1,),
        in_specs=[
            pl.BlockSpec((N_NODES, 128), _rep0),
            _wspec((128, 128)), _wspec((1, 128)),
            _wspec((128, 128)), _wspec((128, 128)), _wspec((128, 128)),
        ],
        out_specs=[
            pl.BlockSpec((N_NODES, 128), _rep0),
            pl.BlockSpec((N_NODES, 256), _rep0),
            pl.BlockSpec((N_NODES, 128), _rep0),
        ],
        out_shape=[
            jax.ShapeDtypeStruct((N_NODES, 128), jnp.float32),
            jax.ShapeDtypeStruct((N_NODES, 256), jnp.float32),
            jax.ShapeDtypeStruct((N_NODES, 128), jnp.float32),
        ],
    )(h_V, emb["w"], emb["b"].reshape(1, 128), wsA, wdA, wsV)


def _node_update1(p, d, h_V, lp):
    n_pad = p.shape[1]

    def body(p_ref, d_ref, hv_ref, wow, wob, g0, b0, x1_ref):
        agg = (p_ref[0, 0:N_NODES, :] + p_ref[1, 0:N_NODES, :]) / (
            d_ref[0, 0:N_NODES, :] + d_ref[1, 0:N_NODES, :] + 1e-9)
        y = hv_ref[...] + agg @ wow[...] + wob[...]
        mu = jnp.mean(y, 0, keepdims=True)
        var = jnp.mean(y * y, 0, keepdims=True) - mu * mu
        x1_ref[...] = (y - mu) * lax.rsqrt(var + 1e-5) * g0[...] + b0[...]

    return pl.pallas_call(
        body,
        grid=(1,),
        in_specs=[
            pl.BlockSpec((_NC, n_pad, 128), lambda i: (0, 0, 0)),
            pl.BlockSpec((_NC, n_pad, 128), lambda i: (0, 0, 0)),
            pl.BlockSpec((N_NODES, 128), _rep0),
            _wspec((128, 128)), _wspec((1, 128)),
            _wspec((1, 128)), _wspec((1, 128)),
        ],
        out_specs=pl.BlockSpec((N_NODES, 128), _rep0),
        out_shape=jax.ShapeDtypeStruct((N_NODES, 128), jnp.float32),
    )(p, d, h_V,
      lp["w_o"]["w"], lp["w_o"]["b"].reshape(1, 128),
      lp["bn0"]["g"].reshape(1, 128), lp["bn0"]["b"].reshape(1, 128))


def _node_update2(x1, lp):
    D1, D2 = lp["dense"]
    CH = 2000

    def body(x1_ref, d1w, d1b, d2w, d2b, g1, b1, v2_ref):
        for c in range(N_NODES // CH):
            xb = x1_ref[pl.ds(c * CH, CH), :]
            mid = jax.nn.gelu(xb @ d1w[...] + d1b[...])
            v2_ref[pl.ds(c * CH, CH), :] = xb + mid @ d2w[...] + d2b[...]
        y = v2_ref[...]
        mu = jnp.mean(y, 0, keepdims=True)
        var = jnp.mean(y * y, 0, keepdims=True) - mu * mu
        v2_ref[...] = (y - mu) * lax.rsqrt(var + 1e-5) * g1[...] + b1[...]

    return pl.pallas_call(
        body,
        grid=(1,),
        in_specs=[
            pl.BlockSpec((N_NODES, 128), _rep0),
            _wspec((128, 512)), _wspec((1, 512)),
            _wspec((512, 128)), _wspec((1, 128)),
            _wspec((1, 128)), _wspec((1, 128)),
        ],
        out_specs=pl.BlockSpec((N_NODES, 128), _rep0),
        out_shape=jax.ShapeDtypeStruct((N_NODES, 128), jnp.float32),
    )(x1, D1["w"], D1["b"].reshape(1, 512), D2["w"], D2["b"].reshape(1, 128),
      lp["bn1"]["g"].reshape(1, 128), lp["bn1"]["b"].reshape(1, 128))


def _node_update3(v2, bid2, lp, next_lp):
    C1, C2, C3 = lp["ctx_g"]
    wsu = lp["w11"]["w"][0:128]
    wdu = lp["w11"]["w"][256:384]
    last = next_lp is None
    if not last:
        wsA = next_lp["att_bias"][0]["w"][256:384]
        wdA = next_lp["att_bias"][0]["w"][0:128]
        wsV = next_lp["w_v"][0]["w"][128:256]
    else:
        wsA = wdA = wsV = jnp.zeros((128, 128), jnp.float32)

    def body(v2_ref, bid_ref, c1w, c1b, c2w, c2b, c3w, c3b,
             wsu_, wdu_, wsa, wda, wsv,
             hg_ref, pu_ref, pd_ref, g1_ref, g2_ref):
        v2b = v2_ref[...]
        oh = jnp.where(
            bid_ref[...] == lax.broadcasted_iota(jnp.int32, (N_NODES, 8), 1),
            1.0, 0.0)
        csum = lax.dot_general(oh, v2b, (((0,), (0,)), ((), ())))
        ones = jnp.full((N_NODES, 1), 1.0, jnp.float32)
        cnt = lax.dot_general(oh, ones, (((0,), (0,)), ((), ())))
        c_V = csum / (cnt + 1e-9)
        gm1 = jax.nn.relu(c_V @ c1w[...] + c1b[...])
        gm2 = jax.nn.relu(gm1 @ c2w[...] + c2b[...])
        gate = jax.nn.sigmoid(gm2 @ c3w[...] + c3b[...])
        hg = v2b * (oh @ gate)
        hg_ref[...] = hg
        pu_ref[...] = v2b @ wsu_[...]
        pd_ref[...] = v2b @ wdu_[...]
        if not last:
            g1_ref[:, 0:128] = hg @ wsa[...]
            g1_ref[:, 128:256] = hg @ wsv[...]
            g2_ref[...] = hg @ wda[...]
        else:
            g1_ref[...] = jnp.zeros((N_NODES, 256), jnp.float32)
            g2_ref[...] = jnp.zeros((N_NODES, 128), jnp.float32)

    return pl.pallas_call(
        body,
        grid=(1,),
        in_specs=[
            pl.BlockSpec((N_NODES, 128), _rep0),
            pl.BlockSpec((N_NODES, 1), _rep0),
            _wspec((128, 128)), _wspec((1, 128)),
            _wspec((128, 128)), _wspec((1, 128)),
            _wspec((128, 128)), _wspec((1, 128)),
            _wspec((128, 128)), _wspec((128, 128)),
            _wspec((128, 128)), _wspec((128, 128)), _wspec((128, 128)),
        ],
        out_specs=[
            pl.BlockSpec((N_NODES, 128), _rep0),
            pl.BlockSpec((N_NODES, 128), _rep0),
            pl.BlockSpec((N_NODES, 128), _rep0),
            pl.BlockSpec((N_NODES, 256), _rep0),
            pl.BlockSpec((N_NODES, 128), _rep0),
        ],
        out_shape=[
            jax.ShapeDtypeStruct((N_NODES, 128), jnp.float32),
            jax.ShapeDtypeStruct((N_NODES, 128), jnp.float32),
            jax.ShapeDtypeStruct((N_NODES, 128), jnp.float32),
            jax.ShapeDtypeStruct((N_NODES, 256), jnp.float32),
            jax.ShapeDtypeStruct((N_NODES, 128), jnp.float32),
        ],
    )(v2, bid2,
      C1["w"], C1["b"].reshape(1, 128),
      C2["w"], C2["b"].reshape(1, 128),
      C3["w"], C3["b"].reshape(1, 128),
      wsu, wdu, wsA, wdA, wsV)


def _readout(hg, params):
    V = params["readout"]["b"].shape[0]

    def body(hg_ref, wr, br, out_ref):
        z = hg_ref[...] @ wr[...] + br[...]
        zmax = jnp.max(z, 1, keepdims=True)
        zc = z - zmax
        lse = jnp.log(jnp.sum(jnp.exp(zc), 1, keepdims=True))
        out_ref[...] = zc - lse

    return pl.pallas_call(
        body,
        grid=(1,),
        in_specs=[
            pl.BlockSpec((N_NODES, 128), _rep0),
            _wspec((128, V)), _wspec((1, V)),
        ],
        out_specs=pl.BlockSpec((N_NODES, V), _rep0),
        out_shape=jax.ShapeDtypeStruct((N_NODES, V), jnp.float32),
    )(hg, params["readout"]["w"], params["readout"]["b"].reshape(1, V))


def kernel(h_V, h_E, params, edge_idx, batch_id):
    src3 = edge_idx[0].reshape(_NW, -1, 80)
    dst3 = edge_idx[1].reshape(_NW, -1, 80)
    dst3s = jnp.pad(
        edge_idx[1].reshape(_NW, -1, 5, 40), ((0, 0), (0, 0), (0, 3), (0, 0)))
    bid2 = batch_id.reshape(N_NODES, 1)
    layers = params["layers"]
    n_emb = params["node_embed"]
    e_emb = params["edge_embed"]

    hv0, g1_tab, g2_tab = _node_embed_proj(h_V, params)
    hv = hv0
    xe = h_E
    stats = jnp.zeros((2, 128), jnp.float32)
    ones128 = jnp.ones((1, 128), jnp.float32)
    zeros128 = jnp.zeros((1, 128), jnp.float32)
    bn_g, bn_b = ones128, zeros128

    for li, lp in enumerate(layers):
        first = li == 0
        last = li == len(layers) - 1
        g1 = _sc_gather(g1_tab, src3)
        g2 = _sc_gather(g2_tab, dst3)
        msgu, ewf = _edge_attention(xe, g1, g2, lp, e_emb, stats, bn_g, bn_b, first)
        p, d = _sc_scatter_add2(msgu, ewf, dst3s, N_NODES)
        x1 = _node_update1(p, d, hv, lp)
        v2 = _node_update2(x1, lp)
        hg, pu, pd, g1n, g2n = _node_update3(
            v2, bid2, lp, None if last else layers[li + 1])
        if not last:
            g3 = _sc_gather(pu, src3)
            g4 = _sc_gather(pd, dst3)
            xe, stats = _edge_update(xe, g3, g4, lp, e_emb, stats, bn_g, bn_b, first)
            bn_g = lp["bn_e"]["g"].reshape(1, 128)
            bn_b = lp["bn_e"]["b"].reshape(1, 128)
            g1_tab, g2_tab = g1n, g2n
        hv = hg
    return _readout(hv, params)


# speedup vs baseline: 5.6103x; 1.1020x over previous
import functools

import jax
import jax.numpy as jnp
from jax import lax
from jax.experimental import pallas as pl
from jax.experimental.pallas import tpu as pltpu
from jax.experimental.pallas import tpu_sc as plsc

N_NODES = 10000
N_EDGES = 320000
HIDDEN = 128
HEADS = 4
N_GRAPHS = 8

_NC = 2
_NS = 16
_NW = _NC * _NS

_BE = 3200
_NBLK = N_EDGES // _BE


def _sc_gather(table, idx3, chunk=80, k=5):
    row_shape = table.shape[1:]
    dt = table.dtype
    NW_, nch, C = idx3.shape
    assert NW_ == _NW and C == chunk and nch % k == 0
    per_w = nch * C
    E = _NW * per_w
    mesh = plsc.VectorSubcoreMesh(
        core_axis_name="c", subcore_axis_name="s", num_cores=_NC, num_subcores=_NS
    )

    @functools.partial(
        pl.kernel,
        mesh=mesh,
        out_type=jax.ShapeDtypeStruct((E,) + row_shape, dt),
        scratch_types=[
            pltpu.VMEM((nch, C), jnp.int32),
            pltpu.VMEM((k, C) + row_shape, dt),
            pltpu.SemaphoreType.DMA,
            pltpu.SemaphoreType.DMA,
        ],
    )
    def kk(table_hbm, idx_hbm, out_hbm, idx_v, rows_v, sem_g, sem_w):
        wid = lax.axis_index("s") * _NC + lax.axis_index("c")
        base = wid * per_w
        pltpu.sync_copy(idx_hbm.at[wid], idx_v)

        def sup(sj, carry):
            j0 = sj * k
            gds = [
                pltpu.async_copy(
                    table_hbm.at[idx_v.at[j0 + b]], rows_v.at[b], sem_g)
                for b in range(k)
            ]
            wds = []
            for b in range(k):
                gds[b].wait()
                wds.append(pltpu.async_copy(
                    rows_v.at[b],
                    out_hbm.at[pl.ds(base + (j0 + b) * C, C)], sem_w))
            for wd in wds:
                wd.wait()
            return carry

        lax.fori_loop(0, nch // k, sup, 0)

    return kk(table, idx3)


def _sc_scatter_add2(vals_a, vals_b, idx4, n, chunk=40, k=5):
    E, D = vals_a.shape
    NW_, nsup, k8, C = idx4.shape
    assert NW_ == _NW and C == chunk and k8 == 8
    per_w = nsup * k * C
    assert per_w * _NW == E
    n_pad = ((n + 8 * _NS - 1) // (8 * _NS)) * (8 * _NS)
    rows_per_sub = n_pad // _NS
    mesh = plsc.VectorSubcoreMesh(
        core_axis_name="c", subcore_axis_name="s", num_cores=_NC, num_subcores=_NS
    )

    @functools.partial(
        pl.kernel,
        mesh=mesh,
        out_type=(
            jax.ShapeDtypeStruct((_NC, n_pad, D), jnp.float32),
            jax.ShapeDtypeStruct((_NC, n_pad, D), jnp.float32),
        ),
        scratch_types=[
            pltpu.VMEM((8, C), jnp.int32),
            pltpu.VMEM((k, C, D), jnp.float32),
            pltpu.VMEM_SHARED((n_pad, D), jnp.float32),
            pltpu.SemaphoreType.DMA,
            pltpu.SemaphoreType.DMA,
        ],
    )
    def kk(zeros_hbm, va_hbm, vb_hbm, idx_hbm, outa_hbm, outb_hbm,
           idx_v, vals_v, table_sh, sem_l, sem_s):
        cid = lax.axis_index("c")
        sid = lax.axis_index("s")
        wid = sid * _NC + cid
        srow = sid * rows_per_sub
        base = wid * per_w

        def zero_table():
            pltpu.sync_copy(
                zeros_hbm.at[pl.ds(srow, rows_per_sub)],
                table_sh.at[pl.ds(srow, rows_per_sub)],
            )

        def run(v_hbm, out_hbm):
            def sup(sj, carry):
                j0 = sj * k
                pltpu.sync_copy(idx_hbm.at[wid, sj], idx_v)
                lds = [
                    pltpu.async_copy(
                        v_hbm.at[pl.ds(base + (j0 + b) * C, C)],
                        vals_v.at[b], sem_l)
                    for b in range(k)
                ]
                sds = []
                for b in range(k):
                    lds[b].wait()
                    sds.append(pltpu.async_copy(
                        vals_v.at[b], table_sh.at[idx_v.at[b]],
                        sem_s, add=True))
                for sd in sds:
                    sd.wait()
                return carry

            lax.fori_loop(0, nsup, sup, 0)
            plsc.subcore_barrier()
            pltpu.sync_copy(
                table_sh.at[pl.ds(srow, rows_per_sub)],
                out_hbm.at[cid, pl.ds(srow, rows_per_sub)],
            )

        zero_table()
        plsc.subcore_barrier()
        run(va_hbm, outa_hbm)
        plsc.subcore_barrier()
        zero_table()
        plsc.subcore_barrier()
        run(vb_hbm, outb_hbm)

    return kk(jnp.zeros((n_pad, D), jnp.float32), vals_a, vals_b, idx4)


def _pack_bf16(a, b):
    au = lax.bitcast_convert_type(a.astype(jnp.bfloat16), jnp.uint16)
    bu = lax.bitcast_convert_type(b.astype(jnp.bfloat16), jnp.uint16)
    return (au.astype(jnp.uint32) | (bu.astype(jnp.uint32) << 16)).astype(jnp.int32)


def _fold_bn(stats_ref, g_ref, b_ref):
    inv_n = 1.0 / N_EDGES
    mu = stats_ref[0:1, :] * inv_n
    var = stats_ref[1:2, :] * inv_n - mu * mu
    sc = g_ref[...] * lax.rsqrt(var + 1e-5)
    sh = b_ref[...] - mu * sc
    return sc, sh


def _edge_blk(i):
    return (i, 0)


def _rep0(i):
    return (0, 0)


def _wspec(shape):
    return pl.BlockSpec(shape, _rep0)


def _edge_attention(xe, g1, g2, lp, emb, stats, bn_g, bn_b, first):
    A1, A2, A3 = lp["att_bias"]
    V1, V2, V3 = lp["w_v"]
    A1we = A1["w"][128:256]
    V1we = V1["w"][0:128]
    a3w8 = jnp.pad(A3["w"], ((0, 0), (0, 4)))
    a3b8 = jnp.pad(A3["b"], (0, 4)).reshape(1, 8)

    def body(xe_ref, g1_ref, g2_ref, a1we, a1b, a2w, a2b, a3w, a3b,
             v1we, v1b, v2w, v2b, v3w, v3b, embw, embb, stats_ref, gref, bref,
             msg_ref, ewf_ref):
        if first:
            wa = embw[...] @ a1we[...]
            ba = embb[...] @ a1we[...] + a1b[...]
            wv = embw[...] @ v1we[...]
            bv = embb[...] @ v1we[...] + v1b[...]
        else:
            sc, sh = _fold_bn(stats_ref, gref, bref)
            wa = a1we[...] * sc.reshape(128, 1)
            ba = sh @ a1we[...] + a1b[...]
            wv = v1we[...] * sc.reshape(128, 1)
            bv = sh @ v1we[...] + v1b[...]
        xe_b = xe_ref[...]
        g1p = g1_ref[...]
        g1a = lax.bitcast_convert_type(
            (g1p & 0xFFFF).astype(jnp.uint16), jnp.bfloat16).astype(jnp.float32)
        t1 = jax.nn.gelu(xe_b @ wa + g1a + g2_ref[...] + ba)
        t2 = jax.nn.gelu(t1 @ a2w[...] + a2b[...])
        w8 = t2 @ a3w[...] + a3b[...]
        ew8 = jnp.exp(w8)
        li = lax.broadcasted_iota(jnp.int32, (8, 128), 1) // 32
        ri = lax.broadcasted_iota(jnp.int32, (8, 128), 0)
        rep = jnp.where((li == ri) & (ri < HEADS), 1.0, 0.0)
        ewf = ew8 @ rep
        g1v = lax.bitcast_convert_type(
            ((g1p >> 16) & 0xFFFF).astype(jnp.uint16),
            jnp.bfloat16).astype(jnp.float32)
        u1 = jax.nn.gelu(xe_b @ wv + g1v + bv)
        u2 = jax.nn.gelu(u1 @ v2w[...] + v2b[...])
        vm = u2 @ v3w[...] + v3b[...]
        ewf_ref[...] = ewf
        msg_ref[...] = ewf * vm

    return pl.pallas_call(
        body,
        grid=(_NBLK,),
        in_specs=[
            pl.BlockSpec((_BE, 128), _edge_blk),
            pl.BlockSpec((_BE, 128), _edge_blk),
            pl.BlockSpec((_BE, 128), _edge_blk),
            _wspec((128, 128)), _wspec((1, 128)),
            _wspec((128, 128)), _wspec((1, 128)),
            _wspec((128, 8)), _wspec((1, 8)),
            _wspec((128, 128)), _wspec((1, 128)),
            _wspec((128, 128)), _wspec((1, 128)),
            _wspec((128, 128)), _wspec((1, 128)),
            _wspec((128, 128)), _wspec((1, 128)),
            _wspec((2, 128)), _wspec((1, 128)), _wspec((1, 128)),
        ],
        out_specs=[
            pl.BlockSpec((_BE, 128), _edge_blk),
            pl.BlockSpec((_BE, 128), _edge_blk),
        ],
        out_shape=[
            jax.ShapeDtypeStruct((N_EDGES, 128), jnp.float32),
            jax.ShapeDtypeStruct((N_EDGES, 128), jnp.float32),
        ],
    )(xe, g1, g2,
      A1we, A1["b"].reshape(1, 128),
      A2["w"], A2["b"].reshape(1, 128),
      a3w8, a3b8,
      V1we, V1["b"].reshape(1, 128),
      V2["w"], V2["b"].reshape(1, 128),
      V3["w"], V3["b"].reshape(1, 128),
      emb["w"], emb["b"].reshape(1, 128),
      stats, bn_g, bn_b)


def _edge_update(xe, g3, g4, lp, emb, stats, bn_g, bn_b, first):
    W11, W12, W13 = lp["w11"], lp["w12"], lp["w13"]
    W11e = W11["w"][128:256]

    def body(xe_ref, g3_ref, g4_ref, w11e, b11, w12w, b12, w13w, b13,
             embw, embb, stats_ref, gref, bref, x_ref, st_ref):
        xe_b = xe_ref[...]
        if first:
            we = embw[...] @ w11e[...]
            be = embb[...] @ w11e[...] + b11[...]
            he = xe_b @ embw[...] + embb[...]
        else:
            sc, sh = _fold_bn(stats_ref, gref, bref)
            we = w11e[...] * sc.reshape(128, 1)
            be = sh @ w11e[...] + b11[...]
            he = xe_b * sc + sh
        q1 = jax.nn.gelu(xe_b @ we + g3_ref[...] + g4_ref[...] + be)
        q2 = jax.nn.gelu(q1 @ w12w[...] + b12[...])
        m = q2 @ w13w[...] + b13[...]
        xn = he + m
        x_ref[...] = xn
        s = jnp.sum(xn, 0, keepdims=True)
        q = jnp.sum(xn * xn, 0, keepdims=True)
        i = pl.program_id(0)

        @pl.when(i == 0)
        def _():
            st_ref[0:1, :] = s
            st_ref[1:2, :] = q

        @pl.when(i != 0)
        def _():
            st_ref[0:1, :] = st_ref[0:1, :] + s
            st_ref[1:2, :] = st_ref[1:2, :] + q

    return pl.pallas_call(
        body,
        grid=(_NBLK,),
        in_specs=[
            pl.BlockSpec((_BE, 128), _edge_blk),
            pl.BlockSpec((_BE, 128), _edge_blk),
            pl.BlockSpec((_BE, 128), _edge_blk),
            _wspec((128, 128)), _wspec((1, 128)),
            _wspec((128, 128)), _wspec((1, 128)),
            _wspec((128, 128)), _wspec((1, 128)),
            _wspec((128, 128)), _wspec((1, 128)),
            _wspec((2, 128)), _wspec((1, 128)), _wspec((1, 128)),
        ],
        out_specs=[
            pl.BlockSpec((_BE, 128), _edge_blk),
            pl.BlockSpec((2, 128), _rep0),
        ],
        out_shape=[
            jax.ShapeDtypeStruct((N_EDGES, 128), jnp.float32),
            jax.ShapeDtypeStruct((2, 128), jnp.float32),
        ],
    )(xe, g3, g4,
      W11e, W11["b"].reshape(1, 128),
      W12["w"], W12["b"].reshape(1, 128),
      W13["w"], W13["b"].reshape(1, 128),
      emb["w"], emb["b"].reshape(1, 128),
      stats, bn_g, bn_b)


def _node_embed_proj(h_V, params):
    lp = params["layers"][0]
    emb = params["node_embed"]
    wsA = lp["att_bias"][0]["w"][256:384]
    wdA = lp["att_bias"][0]["w"][0:128]
    wsV = lp["w_v"][0]["w"][128:256]

    def body(hv_ref, embw, embb, wsa, wda, wsv, hv0_ref, g1_ref, g2_ref):
        hv0 = hv_ref[...] @ embw[...] + embb[...]
        hv0_ref[...] = hv0
        g1_ref[...] = _pack_bf16(hv0 @ wsa[...], hv0 @ wsv[...])
        g2_ref[...] = hv0 @ wda[...]

    return pl.pallas_call(
        body,
        grid=(1,),
        in_specs=[
            pl.BlockSpec((N_NODES, 128), _rep0),
            _wspec((128, 128)), _wspec((1, 128)),
            _wspec((128, 128)), _wspec((128, 128)), _wspec((128, 128)),
        ],
        out_specs=[
            pl.BlockSpec((N_NODES, 128), _rep0),
            pl.BlockSpec((N_NODES, 128), _rep0),
            pl.BlockSpec((N_NODES, 128), _rep0),
        ],
        out_shape=[
            jax.ShapeDtypeStruct((N_NODES, 128), jnp.float32),
            jax.ShapeDtypeStruct((N_NODES, 128), jnp.int32),
            jax.ShapeDtypeStruct((N_NODES, 128), jnp.float32),
        ],
    )(h_V, emb["w"], emb["b"].reshape(1, 128), wsA, wdA, wsV)


def _node_update1(p, d, h_V, lp):
    n_pad = p.shape[1]

    def body(p_ref, d_ref, hv_ref, wow, wob, g0, b0, x1_ref):
        agg = (p_ref[0, 0:N_NODES, :] + p_ref[1, 0:N_NODES, :]) / (
            d_ref[0, 0:N_NODES, :] + d_ref[1, 0:N_NODES, :] + 1e-9)
        y = hv_ref[...] + agg @ wow[...] + wob[...]
        mu = jnp.mean(y, 0, keepdims=True)
        var = jnp.mean(y * y, 0, keepdims=True) - mu * mu
        x1_ref[...] = (y - mu) * lax.rsqrt(var + 1e-5) * g0[...] + b0[...]

    return pl.pallas_call(
        body,
        grid=(1,),
        in_specs=[
            pl.BlockSpec((_NC, n_pad, 128), lambda i: (0, 0, 0)),
            pl.BlockSpec((_NC, n_pad, 128), lambda i: (0, 0, 0)),
            pl.BlockSpec((N_NODES, 128), _rep0),
            _wspec((128, 128)), _wspec((1, 128)),
            _wspec((1, 128)), _wspec((1, 128)),
        ],
        out_specs=pl.BlockSpec((N_NODES, 128), _rep0),
        out_shape=jax.ShapeDtypeStruct((N_NODES, 128), jnp.float32),
    )(p, d, h_V,
      lp["w_o"]["w"], lp["w_o"]["b"].reshape(1, 128),
      lp["bn0"]["g"].reshape(1, 128), lp["bn0"]["b"].reshape(1, 128))


def _node_update2(x1, lp):
    D1, D2 = lp["dense"]
    CH = 2000

    def body(x1_ref, d1w, d1b, d2w, d2b, g1, b1, v2_ref):
        for c in range(N_NODES // CH):
            xb = x1_ref[pl.ds(c * CH, CH), :]
            mid = jax.nn.gelu(xb @ d1w[...] + d1b[...])
            v2_ref[pl.ds(c * CH, CH), :] = xb + mid @ d2w[...] + d2b[...]
        y = v2_ref[...]
        mu = jnp.mean(y, 0, keepdims=True)
        var = jnp.mean(y * y, 0, keepdims=True) - mu * mu
        v2_ref[...] = (y - mu) * lax.rsqrt(var + 1e-5) * g1[...] + b1[...]

    return pl.pallas_call(
        body,
        grid=(1,),
        in_specs=[
            pl.BlockSpec((N_NODES, 128), _rep0),
            _wspec((128, 512)), _wspec((1, 512)),
            _wspec((512, 128)), _wspec((1, 128)),
            _wspec((1, 128)), _wspec((1, 128)),
        ],
        out_specs=pl.BlockSpec((N_NODES, 128), _rep0),
        out_shape=jax.ShapeDtypeStruct((N_NODES, 128), jnp.float32),
    )(x1, D1["w"], D1["b"].reshape(1, 512), D2["w"], D2["b"].reshape(1, 128),
      lp["bn1"]["g"].reshape(1, 128), lp["bn1"]["b"].reshape(1, 128))


def _node_update3(v2, bid2, lp, next_lp):
    C1, C2, C3 = lp["ctx_g"]
    wsu = lp["w11"]["w"][0:128]
    wdu = lp["w11"]["w"][256:384]
    last = next_lp is None
    if not last:
        wsA = next_lp["att_bias"][0]["w"][256:384]
        wdA = next_lp["att_bias"][0]["w"][0:128]
        wsV = next_lp["w_v"][0]["w"][128:256]
    else:
        wsA = wdA = wsV = jnp.zeros((128, 128), jnp.float32)

    def body(v2_ref, bid_ref, c1w, c1b, c2w, c2b, c3w, c3b,
             wsu_, wdu_, wsa, wda, wsv,
             hg_ref, pu_ref, pd_ref, g1_ref, g2_ref):
        v2b = v2_ref[...]
        oh = jnp.where(
            bid_ref[...] == lax.broadcasted_iota(jnp.int32, (N_NODES, 8), 1),
            1.0, 0.0)
        csum = lax.dot_general(oh, v2b, (((0,), (0,)), ((), ())))
        ones = jnp.full((N_NODES, 1), 1.0, jnp.float32)
        cnt = lax.dot_general(oh, ones, (((0,), (0,)), ((), ())))
        c_V = csum / (cnt + 1e-9)
        gm1 = jax.nn.relu(c_V @ c1w[...] + c1b[...])
        gm2 = jax.nn.relu(gm1 @ c2w[...] + c2b[...])
        gate = jax.nn.sigmoid(gm2 @ c3w[...] + c3b[...])
        hg = v2b * (oh @ gate)
        hg_ref[...] = hg
        pu_ref[...] = v2b @ wsu_[...]
        pd_ref[...] = v2b @ wdu_[...]
        if not last:
            g1_ref[...] = _pack_bf16(hg @ wsa[...], hg @ wsv[...])
            g2_ref[...] = hg @ wda[...]
        else:
            g1_ref[...] = jnp.zeros((N_NODES, 128), jnp.int32)
            g2_ref[...] = jnp.zeros((N_NODES, 128), jnp.float32)

    return pl.pallas_call(
        body,
        grid=(1,),
        in_specs=[
            pl.BlockSpec((N_NODES, 128), _rep0),
            pl.BlockSpec((N_NODES, 1), _rep0),
            _wspec((128, 128)), _wspec((1, 128)),
            _wspec((128, 128)), _wspec((1, 128)),
            _wspec((128, 128)), _wspec((1, 128)),
            _wspec((128, 128)), _wspec((128, 128)),
            _wspec((128, 128)), _wspec((128, 128)), _wspec((128, 128)),
        ],
        out_specs=[
            pl.BlockSpec((N_NODES, 128), _rep0),
            pl.BlockSpec((N_NODES, 128), _rep0),
            pl.BlockSpec((N_NODES, 128), _rep0),
            pl.BlockSpec((N_NODES, 128), _rep0),
            pl.BlockSpec((N_NODES, 128), _rep0),
        ],
        out_shape=[
            jax.ShapeDtypeStruct((N_NODES, 128), jnp.float32),
            jax.ShapeDtypeStruct((N_NODES, 128), jnp.float32),
            jax.ShapeDtypeStruct((N_NODES, 128), jnp.float32),
            jax.ShapeDtypeStruct((N_NODES, 128), jnp.int32),
            jax.ShapeDtypeStruct((N_NODES, 128), jnp.float32),
        ],
    )(v2, bid2,
      C1["w"], C1["b"].reshape(1, 128),
      C2["w"], C2["b"].reshape(1, 128),
      C3["w"], C3["b"].reshape(1, 128),
      wsu, wdu, wsA, wdA, wsV)


def _readout(hg, params):
    V = params["readout"]["b"].shape[0]

    def body(hg_ref, wr, br, out_ref):
        z = hg_ref[...] @ wr[...] + br[...]
        zmax = jnp.max(z, 1, keepdims=True)
        zc = z - zmax
        lse = jnp.log(jnp.sum(jnp.exp(zc), 1, keepdims=True))
        out_ref[...] = zc - lse

    return pl.pallas_call(
        body,
        grid=(1,),
        in_specs=[
            pl.BlockSpec((N_NODES, 128), _rep0),
            _wspec((128, V)), _wspec((1, V)),
        ],
        out_specs=pl.BlockSpec((N_NODES, V), _rep0),
        out_shape=jax.ShapeDtypeStruct((N_NODES, V), jnp.float32),
    )(hg, params["readout"]["w"], params["readout"]["b"].reshape(1, V))


def kernel(h_V, h_E, params, edge_idx, batch_id):
    src3 = edge_idx[0].reshape(_NW, -1, 80)
    dst3 = edge_idx[1].reshape(_NW, -1, 80)
    dst3s = jnp.pad(
        edge_idx[1].reshape(_NW, -1, 5, 40), ((0, 0), (0, 0), (0, 3), (0, 0)))
    bid2 = batch_id.reshape(N_NODES, 1)
    layers = params["layers"]
    n_emb = params["node_embed"]
    e_emb = params["edge_embed"]

    hv0, g1_tab, g2_tab = _node_embed_proj(h_V, params)
    hv = hv0
    xe = h_E
    stats = jnp.zeros((2, 128), jnp.float32)
    ones128 = jnp.ones((1, 128), jnp.float32)
    zeros128 = jnp.zeros((1, 128), jnp.float32)
    bn_g, bn_b = ones128, zeros128

    for li, lp in enumerate(layers):
        first = li == 0
        last = li == len(layers) - 1
        g1 = _sc_gather(g1_tab, src3)
        g2 = _sc_gather(g2_tab, dst3)
        msgu, ewf = _edge_attention(xe, g1, g2, lp, e_emb, stats, bn_g, bn_b, first)
        p, d = _sc_scatter_add2(msgu, ewf, dst3s, N_NODES)
        x1 = _node_update1(p, d, hv, lp)
        v2 = _node_update2(x1, lp)
        hg, pu, pd, g1n, g2n = _node_update3(
            v2, bid2, lp, None if last else layers[li + 1])
        if not last:
            g3 = _sc_gather(pu, src3)
            g4 = _sc_gather(pd, dst3)
            xe, stats = _edge_update(xe, g3, g4, lp, e_emb, stats, bn_g, bn_b, first)
            bn_g = lp["bn_e"]["g"].reshape(1, 128)
            bn_b = lp["bn_e"]["b"].reshape(1, 128)
            g1_tab, g2_tab = g1n, g2n
        hv = hg
    return _readout(hv, params)
